# baseline (speedup 1.0000x reference)
import functools

import jax
import jax.numpy as jnp
import numpy as np
from jax import lax
from jax.experimental import pallas as pl
from jax.experimental.pallas import tpu as pltpu
from jax.experimental.pallas import tpu_sc as plsc

NC = 2
NS = 16
NW = NC * NS
LANES = 16


def _prelu(x, a):
    return jnp.where(x >= 0, x, a * x)


def _pair_perm(h):
    p = np.empty(h, np.int32)
    q = h // 2
    for w in range(q):
        p[w] = 32 * (w // 16) + w % 16
        p[q + w] = 32 * (w // 16) + 16 + w % 16
    return p


def _pack_pairs(ylo, yhi):
    blo = lax.bitcast_convert_type(ylo, jnp.int32)
    bhi = lax.bitcast_convert_type(yhi, jnp.int32)
    lo16 = lax.shift_right_logical(blo + jnp.int32(0x8000), 16)
    hi16 = (bhi + jnp.int32(0x8000)) & jnp.int32(-65536)
    return hi16 | lo16


_BCAST_DNUMS = lax.GatherDimensionNumbers(
    offset_dims=(), collapsed_slice_dims=(0,), start_index_map=(0,))


def _lane_bcast(vec, jj):
    idx = jnp.full((LANES, 1), jj, jnp.int32)
    return lax.gather(vec, idx, _BCAST_DNUMS, (1,),
                      mode=lax.GatherScatterMode.PROMISE_IN_BOUNDS)


def _bf16_expand(xi):
    lo = lax.bitcast_convert_type(xi << 16, jnp.float32)
    hi = lax.bitcast_convert_type(xi & jnp.int32(-65536), jnp.float32)
    return lo, hi


def _bits(x):
    return lax.bitcast_convert_type(x, jnp.int32)


def _node_mlp_body(ns, nv0, nv1, nv2, w1, b1, a1, w2, b2, a2,
                   tbl0, tbl1, tbl2, so2):
    h = _prelu(jnp.dot(ns[...], w1[...], preferred_element_type=jnp.float32)
               + b1[...], a1[0, 0])
    y = _prelu(jnp.dot(h, w2[...], preferred_element_type=jnp.float32)
               + b2[...], a2[0, 0])
    H = ns.shape[1]
    Q = H // 2
    so1 = y[:, :H]
    so3p = _pack_pairs(y[:, 2 * H:2 * H + Q], y[:, 2 * H + Q:])
    p0 = nv0[...] * so1
    p1 = nv1[...] * so1
    p2 = nv2[...] * so1
    tbl0[...] = jnp.concatenate(
        [_pack_pairs(p0[:, :Q], p0[:, Q:]), so3p], axis=1)
    tbl1[...] = jnp.concatenate(
        [_pack_pairs(p1[:, :Q], p1[:, Q:]), so3p], axis=1)
    tbl2[...] = jnp.concatenate(
        [_pack_pairs(p2[:, :Q], p2[:, Q:]), so3p], axis=1)
    so2[...] = y[:, H:2 * H]


def _node_mlp(ns, nv0, nv1, nv2, w1, b1, a1, w2, b2, a2):
    n, h = ns.shape
    bn = 400
    grid = (n // bn,)
    row = lambda i: (i, 0)
    fixed = lambda i: (0, 0)
    tbl = jax.ShapeDtypeStruct((n, h), jnp.int32)
    return pl.pallas_call(
        _node_mlp_body,
        grid=grid,
        in_specs=[
            pl.BlockSpec((bn, h), row),
            pl.BlockSpec((bn, h), row),
            pl.BlockSpec((bn, h), row),
            pl.BlockSpec((bn, h), row),
            pl.BlockSpec((h, h), fixed),
            pl.BlockSpec((1, h), fixed),
            pl.BlockSpec((1, 1), fixed),
            pl.BlockSpec((h, 3 * h), fixed),
            pl.BlockSpec((1, 3 * h), fixed),
            pl.BlockSpec((1, 1), fixed),
        ],
        out_specs=[pl.BlockSpec((bn, h), row)] * 4,
        out_shape=[tbl] * 3 + [jax.ShapeDtypeStruct((n, h), jnp.float32)],
    )(ns, nv0, nv1, nv2, w1, b1, a1, w2, b2, a2)


def _filter_mlp_body(rbf, wf1, bf1, af1, wf2, bf2, af2, fwv, fw2):
    h = _prelu(jnp.dot(rbf[...], wf1[...], preferred_element_type=jnp.float32)
               + bf1[...], af1[0, 0])
    y = _prelu(jnp.dot(h.astype(jnp.bfloat16),
                       wf2[...].astype(jnp.bfloat16),
                       preferred_element_type=jnp.float32)
               + bf2[...], af2[0, 0])
    H = wf1.shape[1]
    Q = H // 2
    fwv[...] = jnp.concatenate(
        [_pack_pairs(y[:, :Q], y[:, Q:H]),
         _pack_pairs(y[:, 2 * H:2 * H + Q], y[:, 2 * H + Q:])], axis=1)
    fw2[...] = _pack_pairs(y[:, H:H + Q], y[:, H + Q:2 * H])


def _filter_mlp(rbf, wf1, bf1, af1, wf2, bf2, af2, e2):
    e, rb = rbf.shape
    h = wf1.shape[1]
    be = 2520
    grid = (-(-e // be),)
    row = lambda i: (i, 0)
    fixed = lambda i: (0, 0)
    return pl.pallas_call(
        _filter_mlp_body,
        grid=grid,
        in_specs=[
            pl.BlockSpec((be, rb), row),
            pl.BlockSpec((rb, h), fixed),
            pl.BlockSpec((1, h), fixed),
            pl.BlockSpec((1, 1), fixed),
            pl.BlockSpec((h, 3 * h), fixed),
            pl.BlockSpec((1, 3 * h), fixed),
            pl.BlockSpec((1, 1), fixed),
        ],
        out_specs=[pl.BlockSpec((be, h), row),
                   pl.BlockSpec((be, h // 2), row)],
        out_shape=[jax.ShapeDtypeStruct((e2, h), jnp.int32),
                   jax.ShapeDtypeStruct((e2, h // 2), jnp.int32)],
    )(rbf, wf1, bf1, af1, wf2, bf2, af2)


def _sc_scalar_pass(so2, fw2i, src, dst, zeros):
    n, h = so2.shape
    h2 = h // 2
    na = zeros.shape[0]
    e = src.shape[0]
    ew = e // NW
    ch = 80
    nt = ew // ch
    nr = na // NS
    mesh = plsc.VectorSubcoreMesh(core_axis_name="c", subcore_axis_name="s",
                                  num_cores=NC, num_subcores=NS)

    @functools.partial(
        pl.kernel, mesh=mesh,
        out_type=jax.ShapeDtypeStruct((NC * na, h), jnp.float32),
        scratch_types=[
            pltpu.VMEM((ch,), jnp.int32), pltpu.VMEM((ch,), jnp.int32),
            pltpu.VMEM((ch,), jnp.int32), pltpu.VMEM((ch,), jnp.int32),
            pltpu.VMEM((ch, h), jnp.float32), pltpu.VMEM((ch, h), jnp.float32),
            pltpu.VMEM((ch, h2), jnp.int32), pltpu.VMEM((ch, h2), jnp.int32),
            pltpu.VMEM_SHARED((na, h), jnp.float32),
            pltpu.SemaphoreType.DMA, pltpu.SemaphoreType.DMA,
            pltpu.SemaphoreType.DMA, pltpu.SemaphoreType.DMA,
        ],
    )
    def k(so2_hbm, fw2_hbm, src_hbm, dst_hbm, z_hbm, out_hbm,
          dst0, dst1, src0, src1, r0, r1, f0, f1, acc,
          sl0, sl1, sg0, sg1):
        core = lax.axis_index("c")
        sid = lax.axis_index("s")
        wid = sid * NC + core
        dstv = (dst0, dst1)
        srcv = (src0, src1)
        rv = (r0, r1)
        fv = (f0, f1)
        slin = (sl0, sl1)
        sgat = (sg0, sg1)

        pltpu.sync_copy(z_hbm.at[pl.ds(sid * nr, nr)],
                        acc.at[pl.ds(sid * nr, nr)])
        plsc.subcore_barrier()

        def issue_linear(t, b):
            base = wid * ew + t * ch
            pltpu.async_copy(dst_hbm.at[pl.ds(base, ch)], dstv[b], slin[b])
            pltpu.async_copy(src_hbm.at[pl.ds(base, ch)], srcv[b], slin[b])
            pltpu.async_copy(fw2_hbm.at[pl.ds(base, ch)], fv[b], slin[b])

        def wait_linear(b):
            z = pl.ds(0, ch)
            pltpu.make_async_copy(dst_hbm.at[z], dstv[b], slin[b]).wait()
            pltpu.make_async_copy(src_hbm.at[z], srcv[b], slin[b]).wait()
            pltpu.make_async_copy(fw2_hbm.at[z], fv[b], slin[b]).wait()

        def issue_gather(b):
            pltpu.async_copy(so2_hbm.at[dstv[b]], rv[b], sgat[b])

        def wait_gather(b):
            pltpu.make_async_copy(so2_hbm.at[dstv[b]], rv[b], sgat[b]).wait()

        def compute_scatter(b):
            def body(i, _):
                for j in range(h // 32):
                    fl, fh = _bf16_expand(fv[b][i, pl.ds(LANES * j, LANES)])
                    ml = rv[b][i, pl.ds(32 * j, LANES)] * fl
                    mh = rv[b][i, pl.ds(32 * j + LANES, LANES)] * fh
                    rv[b][i, pl.ds(32 * j, LANES)] = ml
                    rv[b][i, pl.ds(32 * j + LANES, LANES)] = mh
                return 0
            lax.fori_loop(0, ch, body, 0)
            pltpu.sync_copy(rv[b], acc.at[srcv[b]], add=True)

        issue_linear(0, 0)
        wait_linear(0)
        issue_gather(0)
        issue_linear(1, 1)

        def step(q, _):
            t = 2 * q
            guard = q < nt // 2 - 1
            wait_gather(0)
            wait_linear(1)
            issue_gather(1)
            compute_scatter(0)

            @pl.when(guard)
            def _():
                issue_linear(t + 2, 0)
            wait_gather(1)

            @pl.when(guard)
            def _():
                wait_linear(0)
                issue_gather(0)
            compute_scatter(1)

            @pl.when(guard)
            def _():
                issue_linear(t + 3, 1)
            return 0
        lax.fori_loop(0, nt // 2, step, 0)

        plsc.subcore_barrier()
        pltpu.sync_copy(acc.at[pl.ds(sid * nr, nr)],
                        out_hbm.at[pl.ds(core * na + sid * nr, nr)])

    return k(so2, fw2i, src, dst, zeros)


def _sc_vec_pass(tbl, fwv, diff_c, dist, src, dst, zeros):
    n, hp = tbl.shape
    h = hp
    h2 = h // 2
    na = zeros.shape[0]
    e = src.shape[0]
    ew = e // NW
    ch = 80
    nt = ew // ch
    nr = na // NS
    mesh = plsc.VectorSubcoreMesh(core_axis_name="c", subcore_axis_name="s",
                                  num_cores=NC, num_subcores=NS)

    @functools.partial(
        pl.kernel, mesh=mesh,
        out_type=jax.ShapeDtypeStruct((NC * na, h), jnp.float32),
        scratch_types=[
            pltpu.VMEM((ch,), jnp.int32), pltpu.VMEM((ch,), jnp.int32),
            pltpu.VMEM((ch,), jnp.int32), pltpu.VMEM((ch,), jnp.int32),
            pltpu.VMEM((ch, hp), jnp.int32), pltpu.VMEM((ch, hp), jnp.int32),
            pltpu.VMEM((ch, hp), jnp.float32), pltpu.VMEM((ch, hp), jnp.float32),
            pltpu.VMEM((ch,), jnp.float32), pltpu.VMEM((ch,), jnp.float32),
            pltpu.VMEM((ch,), jnp.float32), pltpu.VMEM((ch,), jnp.float32),
            pltpu.VMEM_SHARED((na, h), jnp.float32),
            pltpu.SemaphoreType.DMA, pltpu.SemaphoreType.DMA,
            pltpu.SemaphoreType.DMA, pltpu.SemaphoreType.DMA,
        ],
    )
    def k(tbl_hbm, fwv_hbm, diff_hbm, dist_hbm, src_hbm, dst_hbm, z_hbm,
          out_hbm, dst0, dst1, src0, src1, g0, g1, f0, f1,
          df0, df1, ds0, ds1, acc, sl0, sl1, sg0, sg1):
        core = lax.axis_index("c")
        sid = lax.axis_index("s")
        wid = sid * NC + core
        dstv = (dst0, dst1)
        srcv = (src0, src1)
        gv = (g0, g1)
        fv = (f0, f1)
        dfv = (df0, df1)
        dsv = (ds0, ds1)
        slin = (sl0, sl1)
        sgat = (sg0, sg1)

        pltpu.sync_copy(z_hbm.at[pl.ds(sid * nr, nr)],
                        acc.at[pl.ds(sid * nr, nr)])
        plsc.subcore_barrier()

        def issue_linear(t, b):
            base = wid * ew + t * ch
            pltpu.async_copy(dst_hbm.at[pl.ds(base, ch)], dstv[b], slin[b])
            pltpu.async_copy(src_hbm.at[pl.ds(base, ch)], srcv[b], slin[b])
            pltpu.async_copy(fwv_hbm.at[pl.ds(base, ch)], fv[b], slin[b])
            pltpu.async_copy(diff_hbm.at[pl.ds(base, ch)], dfv[b], slin[b])
            pltpu.async_copy(dist_hbm.at[pl.ds(base, ch)], dsv[b], slin[b])

        def wait_linear(b):
            z = pl.ds(0, ch)
            pltpu.make_async_copy(dst_hbm.at[z], dstv[b], slin[b]).wait()
            pltpu.make_async_copy(src_hbm.at[z], srcv[b], slin[b]).wait()
            pltpu.make_async_copy(fwv_hbm.at[z], fv[b], slin[b]).wait()
            pltpu.make_async_copy(diff_hbm.at[z], dfv[b], slin[b]).wait()
            pltpu.make_async_copy(dist_hbm.at[z], dsv[b], slin[b]).wait()

        def issue_gather(b):
            pltpu.async_copy(tbl_hbm.at[dstv[b]], gv[b], sgat[b])

        def wait_gather(b):
            pltpu.make_async_copy(tbl_hbm.at[dstv[b]], gv[b], sgat[b]).wait()

        def compute_scatter(b):
            def grp(g, _):
                gsl = pl.ds(LANES * g, LANES)
                udv = dfv[b][gsl] / dsv[b][gsl]
                for jj in range(LANES):
                    i = LANES * g + jj
                    bc = _lane_bcast(udv, jj)
                    res = []
                    for j in range(h // 32):
                        slp = pl.ds(LANES * j, LANES)
                        slq = pl.ds(h2 + LANES * j, LANES)
                        pl_, ph_ = _bf16_expand(gv[b][i, slp])
                        s3l, s3h = _bf16_expand(gv[b][i, slq])
                        f1l, f1h = _bf16_expand(_bits(fv[b][i, slp]))
                        f3l, f3h = _bf16_expand(_bits(fv[b][i, slq]))
                        res.append(pl_ * f1l + bc * (s3l * f3l))
                        res.append(ph_ * f1h + bc * (s3h * f3h))
                    for j in range(h // 32):
                        fv[b][i, pl.ds(32 * j, LANES)] = res[2 * j]
                        fv[b][i, pl.ds(32 * j + LANES, LANES)] = res[2 * j + 1]
                return 0
            lax.fori_loop(0, ch // LANES, grp, 0)
            pltpu.sync_copy(fv[b], acc.at[srcv[b]], add=True)

        issue_linear(0, 0)
        wait_linear(0)
        issue_gather(0)
        issue_linear(1, 1)

        def step(q, _):
            t = 2 * q
            guard = q < nt // 2 - 1
            wait_gather(0)
            wait_linear(1)
            issue_gather(1)
            compute_scatter(0)

            @pl.when(guard)
            def _():
                issue_linear(t + 2, 0)
            wait_gather(1)

            @pl.when(guard)
            def _():
                wait_linear(0)
                issue_gather(0)
            compute_scatter(1)

            @pl.when(guard)
            def _():
                issue_linear(t + 3, 1)
            return 0
        lax.fori_loop(0, nt // 2, step, 0)

        plsc.subcore_barrier()
        pltpu.sync_copy(acc.at[pl.ds(sid * nr, nr)],
                        out_hbm.at[pl.ds(core * na + sid * nr, nr)])

    return k(tbl, fwv, diff_c, dist, src, dst, zeros)


def _final_body(ns, nv, rs, rv0, rv1, rv2, os_, ov):
    os_[...] = ns[...] + rs[0] + rs[1]
    v0 = nv[:, 0, :] + rv0[0] + rv0[1]
    v1 = nv[:, 1, :] + rv1[0] + rv1[1]
    v2 = nv[:, 2, :] + rv2[0] + rv2[1]
    ov[...] = jnp.stack([v0, v1, v2], axis=1)


def _final_add(ns, nv, rs, rv0, rv1, rv2):
    n, h = ns.shape
    bn = 400
    grid = (n // bn,)
    row = lambda i: (i, 0)
    rowv = lambda i: (i, 0, 0)
    row3 = lambda i: (0, i, 0)
    return pl.pallas_call(
        _final_body,
        grid=grid,
        in_specs=[pl.BlockSpec((bn, h), row),
                  pl.BlockSpec((bn, 3, h), rowv)]
                 + [pl.BlockSpec((NC, bn, h), row3)] * 4,
        out_specs=[pl.BlockSpec((bn, h), row),
                   pl.BlockSpec((bn, 3, h), rowv)],
        out_shape=[jax.ShapeDtypeStruct((n, h), jnp.float32),
                   jax.ShapeDtypeStruct((n, 3, h), jnp.float32)],
    )(ns, nv, rs, rv0, rv1, rv2)


def kernel(node_scalar, node_vector, edge, edge_diff, edge_dist, rbf_dist,
           W1, b1, a1, W2, b2, a2, Wf1, bf1, af1, Wf2, bf2, af2):
    n, h = node_scalar.shape
    e = edge.shape[0]
    e2 = -(-e // 5120) * 5120
    pad = e2 - e
    na = -(-(n + 1) // 128) * 128
    perm = _pair_perm(h)
    perm_w2 = np.concatenate([perm, h + np.arange(h, dtype=np.int32),
                              2 * h + perm])
    perm_wf2 = np.concatenate([perm, h + perm, 2 * h + perm])

    src = jnp.concatenate([edge[:, 0].astype(jnp.int32),
                           jnp.full((pad,), n, jnp.int32)])
    dst = jnp.concatenate([edge[:, 1].astype(jnp.int32),
                           jnp.zeros((pad,), jnp.int32)])
    dist_p = jnp.concatenate([edge_dist, jnp.ones((pad,), jnp.float32)])
    diff_p = jnp.concatenate([edge_diff, jnp.zeros((pad, 3), jnp.float32)])
    zeros_acc = jnp.zeros((na, h), jnp.float32)
    nv0 = node_vector[:, 0, :]
    nv1 = node_vector[:, 1, :]
    nv2 = node_vector[:, 2, :]
    nv0p = nv0[:, perm]
    nv1p = nv1[:, perm]
    nv2p = nv2[:, perm]
    W2p = W2[:, perm_w2]
    b2p = b2[perm_w2]
    Wf2p = Wf2[:, perm_wf2]
    bf2p = bf2[perm_wf2]
    d0 = diff_p[:, 0]
    d1 = diff_p[:, 1]
    d2 = diff_p[:, 2]
    b1r = b1.reshape(1, h)
    b2r = b2p.reshape(1, 3 * h)
    bf1r = bf1.reshape(1, h)
    bf2r = bf2p.reshape(1, 3 * h)
    a1r = a1.reshape(1, 1)
    a2r = a2.reshape(1, 1)
    af1r = af1.reshape(1, 1)
    af2r = af2.reshape(1, 1)

    tbl0, tbl1, tbl2, so2 = _node_mlp(node_scalar, nv0p, nv1p, nv2p,
                                      W1, b1r, a1r, W2p, b2r, a2r)
    fwv, fw2 = _filter_mlp(rbf_dist, Wf1, bf1r, af1r, Wf2p, bf2r, af2r, e2)
    fwv = lax.bitcast_convert_type(fwv, jnp.float32)

    rs = _sc_scalar_pass(so2, fw2, src, dst, zeros_acc).reshape(NC, na, h)
    rv0 = _sc_vec_pass(tbl0, fwv, d0, dist_p, src, dst,
                       zeros_acc).reshape(NC, na, h)
    rv1 = _sc_vec_pass(tbl1, fwv, d1, dist_p, src, dst,
                       zeros_acc).reshape(NC, na, h)
    rv2 = _sc_vec_pass(tbl2, fwv, d2, dist_p, src, dst,
                       zeros_acc).reshape(NC, na, h)

    os_, out_vector = _final_add(node_scalar, node_vector, rs, rv0, rv1, rv2)
    return (os_, out_vector)


# speedup vs baseline: 1.0223x; 1.0223x over previous
import functools

import jax
import jax.numpy as jnp
import numpy as np
from jax import lax
from jax.experimental import pallas as pl
from jax.experimental.pallas import tpu as pltpu
from jax.experimental.pallas import tpu_sc as plsc

NC = 2
NS = 16
NW = NC * NS
LANES = 16


def _prelu(x, a):
    return jnp.where(x >= 0, x, a * x)


def _pair_perm(h):
    p = np.empty(h, np.int32)
    q = h // 2
    for w in range(q):
        p[w] = 32 * (w // 16) + w % 16
        p[q + w] = 32 * (w // 16) + 16 + w % 16
    return p


def _pack_pairs(ylo, yhi):
    blo = lax.bitcast_convert_type(ylo, jnp.int32)
    bhi = lax.bitcast_convert_type(yhi, jnp.int32)
    lo16 = lax.shift_right_logical(blo + jnp.int32(0x8000), 16)
    hi16 = (bhi + jnp.int32(0x8000)) & jnp.int32(-65536)
    return hi16 | lo16


_BCAST_DNUMS = lax.GatherDimensionNumbers(
    offset_dims=(), collapsed_slice_dims=(0,), start_index_map=(0,))


def _lane_bcast(vec, jj):
    idx = jnp.full((LANES, 1), jj, jnp.int32)
    return lax.gather(vec, idx, _BCAST_DNUMS, (1,),
                      mode=lax.GatherScatterMode.PROMISE_IN_BOUNDS)


def _bf16_expand(xi):
    lo = lax.bitcast_convert_type(xi << 16, jnp.float32)
    hi = lax.bitcast_convert_type(xi & jnp.int32(-65536), jnp.float32)
    return lo, hi


def _bits(x):
    return lax.bitcast_convert_type(x, jnp.int32)


def _node_mlp_body(ns, nv0, nv1, nv2, w1, b1, a1, w2, b2, a2,
                   tbl0, tbl1, tbl2, so2):
    h = _prelu(jnp.dot(ns[...], w1[...], preferred_element_type=jnp.float32)
               + b1[...], a1[0, 0])
    y = _prelu(jnp.dot(h, w2[...], preferred_element_type=jnp.float32)
               + b2[...], a2[0, 0])
    H = ns.shape[1]
    Q = H // 2
    so1 = y[:, :H]
    so3p = _pack_pairs(y[:, 2 * H:2 * H + Q], y[:, 2 * H + Q:])
    p0 = nv0[...] * so1
    p1 = nv1[...] * so1
    p2 = nv2[...] * so1
    tbl0[...] = jnp.concatenate(
        [_pack_pairs(p0[:, :Q], p0[:, Q:]), so3p], axis=1)
    tbl1[...] = jnp.concatenate(
        [_pack_pairs(p1[:, :Q], p1[:, Q:]), so3p], axis=1)
    tbl2[...] = jnp.concatenate(
        [_pack_pairs(p2[:, :Q], p2[:, Q:]), so3p], axis=1)
    so2[...] = y[:, H:2 * H]


def _node_mlp(ns, nv0, nv1, nv2, w1, b1, a1, w2, b2, a2):
    n, h = ns.shape
    bn = 400
    grid = (n // bn,)
    row = lambda i: (i, 0)
    fixed = lambda i: (0, 0)
    tbl = jax.ShapeDtypeStruct((n, h), jnp.int32)
    return pl.pallas_call(
        _node_mlp_body,
        grid=grid,
        in_specs=[
            pl.BlockSpec((bn, h), row),
            pl.BlockSpec((bn, h), row),
            pl.BlockSpec((bn, h), row),
            pl.BlockSpec((bn, h), row),
            pl.BlockSpec((h, h), fixed),
            pl.BlockSpec((1, h), fixed),
            pl.BlockSpec((1, 1), fixed),
            pl.BlockSpec((h, 3 * h), fixed),
            pl.BlockSpec((1, 3 * h), fixed),
            pl.BlockSpec((1, 1), fixed),
        ],
        out_specs=[pl.BlockSpec((bn, h), row)] * 4,
        out_shape=[tbl] * 3 + [jax.ShapeDtypeStruct((n, h), jnp.float32)],
    )(ns, nv0, nv1, nv2, w1, b1, a1, w2, b2, a2)


def _filter_mlp_body(rbf, wf1, bf1, af1, wf2, bf2, af2, fwv, fw2):
    h = _prelu(jnp.dot(rbf[...], wf1[...], preferred_element_type=jnp.float32)
               + bf1[...], af1[0, 0])
    y = _prelu(jnp.dot(h.astype(jnp.bfloat16),
                       wf2[...].astype(jnp.bfloat16),
                       preferred_element_type=jnp.float32)
               + bf2[...], af2[0, 0])
    H = wf1.shape[1]
    Q = H // 2
    fwv[...] = jnp.concatenate(
        [_pack_pairs(y[:, :Q], y[:, Q:H]),
         _pack_pairs(y[:, 2 * H:2 * H + Q], y[:, 2 * H + Q:])], axis=1)
    fw2[...] = _pack_pairs(y[:, H:H + Q], y[:, H + Q:2 * H])


def _filter_mlp(rbf, wf1, bf1, af1, wf2, bf2, af2, e2):
    e, rb = rbf.shape
    h = wf1.shape[1]
    be = 2520
    grid = (-(-e // be),)
    row = lambda i: (i, 0)
    fixed = lambda i: (0, 0)
    return pl.pallas_call(
        _filter_mlp_body,
        grid=grid,
        in_specs=[
            pl.BlockSpec((be, rb), row),
            pl.BlockSpec((rb, h), fixed),
            pl.BlockSpec((1, h), fixed),
            pl.BlockSpec((1, 1), fixed),
            pl.BlockSpec((h, 3 * h), fixed),
            pl.BlockSpec((1, 3 * h), fixed),
            pl.BlockSpec((1, 1), fixed),
        ],
        out_specs=[pl.BlockSpec((be, h), row),
                   pl.BlockSpec((be, h // 2), row)],
        out_shape=[jax.ShapeDtypeStruct((e2, h), jnp.int32),
                   jax.ShapeDtypeStruct((e2, h // 2), jnp.int32)],
    )(rbf, wf1, bf1, af1, wf2, bf2, af2)


def _sc_scalar_pass(so2, fw2i, src, dst, zeros):
    n, h = so2.shape
    h2 = h // 2
    na = zeros.shape[0]
    e = src.shape[0]
    ew = e // NW
    ch = 80
    nt = ew // ch
    nr = na // NS
    mesh = plsc.VectorSubcoreMesh(core_axis_name="c", subcore_axis_name="s",
                                  num_cores=NC, num_subcores=NS)

    @functools.partial(
        pl.kernel, mesh=mesh,
        out_type=jax.ShapeDtypeStruct((NC * na, h), jnp.float32),
        scratch_types=[
            pltpu.VMEM((ch,), jnp.int32), pltpu.VMEM((ch,), jnp.int32),
            pltpu.VMEM((ch,), jnp.int32), pltpu.VMEM((ch,), jnp.int32),
            pltpu.VMEM((ch, h), jnp.float32), pltpu.VMEM((ch, h), jnp.float32),
            pltpu.VMEM((ch, h2), jnp.int32), pltpu.VMEM((ch, h2), jnp.int32),
            pltpu.VMEM_SHARED((na, h), jnp.float32),
            pltpu.SemaphoreType.DMA, pltpu.SemaphoreType.DMA,
            pltpu.SemaphoreType.DMA, pltpu.SemaphoreType.DMA,
        ],
    )
    def k(so2_hbm, fw2_hbm, src_hbm, dst_hbm, z_hbm, out_hbm,
          dst0, dst1, src0, src1, r0, r1, f0, f1, acc,
          sl0, sl1, sg0, sg1):
        core = lax.axis_index("c")
        sid = lax.axis_index("s")
        wid = sid * NC + core
        dstv = (dst0, dst1)
        srcv = (src0, src1)
        rv = (r0, r1)
        fv = (f0, f1)
        slin = (sl0, sl1)
        sgat = (sg0, sg1)

        pltpu.sync_copy(z_hbm.at[pl.ds(sid * nr, nr)],
                        acc.at[pl.ds(sid * nr, nr)])
        plsc.subcore_barrier()

        def issue_linear(t, b):
            base = wid * ew + t * ch
            pltpu.async_copy(dst_hbm.at[pl.ds(base, ch)], dstv[b], slin[b])
            pltpu.async_copy(src_hbm.at[pl.ds(base, ch)], srcv[b], slin[b])
            pltpu.async_copy(fw2_hbm.at[pl.ds(base, ch)], fv[b], slin[b])

        def wait_linear(b):
            z = pl.ds(0, ch)
            pltpu.make_async_copy(dst_hbm.at[z], dstv[b], slin[b]).wait()
            pltpu.make_async_copy(src_hbm.at[z], srcv[b], slin[b]).wait()
            pltpu.make_async_copy(fw2_hbm.at[z], fv[b], slin[b]).wait()

        def issue_gather(b):
            pltpu.async_copy(so2_hbm.at[dstv[b]], rv[b], sgat[b])

        def wait_gather(b):
            pltpu.make_async_copy(so2_hbm.at[dstv[b]], rv[b], sgat[b]).wait()

        def compute_scatter(b):
            def body(i, _):
                for j in range(h // 32):
                    fl, fh = _bf16_expand(fv[b][i, pl.ds(LANES * j, LANES)])
                    ml = rv[b][i, pl.ds(32 * j, LANES)] * fl
                    mh = rv[b][i, pl.ds(32 * j + LANES, LANES)] * fh
                    rv[b][i, pl.ds(32 * j, LANES)] = ml
                    rv[b][i, pl.ds(32 * j + LANES, LANES)] = mh
                return 0
            lax.fori_loop(0, ch, body, 0)
            pltpu.sync_copy(rv[b], acc.at[srcv[b]], add=True)

        issue_linear(0, 0)
        wait_linear(0)
        issue_gather(0)
        issue_linear(1, 1)

        def step(q, _):
            t = 2 * q
            guard = q < nt // 2 - 1
            wait_gather(0)
            wait_linear(1)
            issue_gather(1)
            compute_scatter(0)

            @pl.when(guard)
            def _():
                issue_linear(t + 2, 0)
            wait_gather(1)

            @pl.when(guard)
            def _():
                wait_linear(0)
                issue_gather(0)
            compute_scatter(1)

            @pl.when(guard)
            def _():
                issue_linear(t + 3, 1)
            return 0
        lax.fori_loop(0, nt // 2, step, 0)

        plsc.subcore_barrier()
        pltpu.sync_copy(acc.at[pl.ds(sid * nr, nr)],
                        out_hbm.at[pl.ds(core * na + sid * nr, nr)])

    return k(so2, fw2i, src, dst, zeros)


def _sc_vec_pass(tbl, fwv, diff_c, dist, src, dst, zeros):
    n, hp = tbl.shape
    h = hp
    h2 = h // 2
    na = zeros.shape[0]
    e = src.shape[0]
    ew = e // NW
    ch = 80
    nt = ew // ch
    nr = na // NS
    mesh = plsc.VectorSubcoreMesh(core_axis_name="c", subcore_axis_name="s",
                                  num_cores=NC, num_subcores=NS)

    @functools.partial(
        pl.kernel, mesh=mesh,
        out_type=jax.ShapeDtypeStruct((NC * na, h), jnp.float32),
        scratch_types=[
            pltpu.VMEM((ch,), jnp.int32), pltpu.VMEM((ch,), jnp.int32),
            pltpu.VMEM((ch,), jnp.int32), pltpu.VMEM((ch,), jnp.int32),
            pltpu.VMEM((ch, hp), jnp.int32), pltpu.VMEM((ch, hp), jnp.int32),
            pltpu.VMEM((ch, hp), jnp.float32), pltpu.VMEM((ch, hp), jnp.float32),
            pltpu.VMEM((ch,), jnp.float32), pltpu.VMEM((ch,), jnp.float32),
            pltpu.VMEM((ch,), jnp.float32), pltpu.VMEM((ch,), jnp.float32),
            pltpu.VMEM_SHARED((na, h), jnp.float32),
            pltpu.SemaphoreType.DMA, pltpu.SemaphoreType.DMA,
            pltpu.SemaphoreType.DMA, pltpu.SemaphoreType.DMA,
        ],
    )
    def k(tbl_hbm, fwv_hbm, diff_hbm, dist_hbm, src_hbm, dst_hbm, z_hbm,
          out_hbm, dst0, dst1, src0, src1, g0, g1, f0, f1,
          df0, df1, ds0, ds1, acc, sl0, sl1, sg0, sg1):
        core = lax.axis_index("c")
        sid = lax.axis_index("s")
        wid = sid * NC + core
        dstv = (dst0, dst1)
        srcv = (src0, src1)
        gv = (g0, g1)
        fv = (f0, f1)
        dfv = (df0, df1)
        dsv = (ds0, ds1)
        slin = (sl0, sl1)
        sgat = (sg0, sg1)

        pltpu.sync_copy(z_hbm.at[pl.ds(sid * nr, nr)],
                        acc.at[pl.ds(sid * nr, nr)])
        plsc.subcore_barrier()

        def issue_linear(t, b):
            base = wid * ew + t * ch
            pltpu.async_copy(dst_hbm.at[pl.ds(base, ch)], dstv[b], slin[b])
            pltpu.async_copy(src_hbm.at[pl.ds(base, ch)], srcv[b], slin[b])
            pltpu.async_copy(fwv_hbm.at[pl.ds(base, ch)], fv[b], slin[b])
            pltpu.async_copy(diff_hbm.at[pl.ds(base, ch)], dfv[b], slin[b])
            pltpu.async_copy(dist_hbm.at[pl.ds(base, ch)], dsv[b], slin[b])

        def wait_linear(b):
            z = pl.ds(0, ch)
            pltpu.make_async_copy(dst_hbm.at[z], dstv[b], slin[b]).wait()
            pltpu.make_async_copy(src_hbm.at[z], srcv[b], slin[b]).wait()
            pltpu.make_async_copy(fwv_hbm.at[z], fv[b], slin[b]).wait()
            pltpu.make_async_copy(diff_hbm.at[z], dfv[b], slin[b]).wait()
            pltpu.make_async_copy(dist_hbm.at[z], dsv[b], slin[b]).wait()

        def issue_gather(b):
            pltpu.async_copy(tbl_hbm.at[dstv[b]], gv[b], sgat[b])

        def wait_gather(b):
            pltpu.make_async_copy(tbl_hbm.at[dstv[b]], gv[b], sgat[b]).wait()

        def compute_scatter(b):
            def grp(g, _):
                gsl = pl.ds(LANES * g, LANES)
                udv = dfv[b][gsl] / dsv[b][gsl]
                for jj in range(LANES):
                    i = LANES * g + jj
                    bc = _lane_bcast(udv, jj)
                    res = []
                    for j in range(h // 32):
                        slp = pl.ds(LANES * j, LANES)
                        slq = pl.ds(h2 + LANES * j, LANES)
                        pl_, ph_ = _bf16_expand(gv[b][i, slp])
                        s3l, s3h = _bf16_expand(gv[b][i, slq])
                        f1l, f1h = _bf16_expand(_bits(fv[b][i, slp]))
                        f3l, f3h = _bf16_expand(_bits(fv[b][i, slq]))
                        res.append(pl_ * f1l + bc * (s3l * f3l))
                        res.append(ph_ * f1h + bc * (s3h * f3h))
                    for j in range(h // 32):
                        fv[b][i, pl.ds(32 * j, LANES)] = res[2 * j]
                        fv[b][i, pl.ds(32 * j + LANES, LANES)] = res[2 * j + 1]
                return 0
            lax.fori_loop(0, ch // LANES, grp, 0)
            pltpu.sync_copy(fv[b], acc.at[srcv[b]], add=True)

        issue_linear(0, 0)
        wait_linear(0)
        issue_gather(0)
        issue_linear(1, 1)

        def step(q, _):
            t = 2 * q
            guard = q < nt // 2 - 1
            wait_gather(0)
            wait_linear(1)
            issue_gather(1)
            compute_scatter(0)

            @pl.when(guard)
            def _():
                issue_linear(t + 2, 0)
            wait_gather(1)

            @pl.when(guard)
            def _():
                wait_linear(0)
                issue_gather(0)
            compute_scatter(1)

            @pl.when(guard)
            def _():
                issue_linear(t + 3, 1)
            return 0
        lax.fori_loop(0, nt // 2, step, 0)

        plsc.subcore_barrier()
        pltpu.sync_copy(acc.at[pl.ds(sid * nr, nr)],
                        out_hbm.at[pl.ds(core * na + sid * nr, nr)])

    return k(tbl, fwv, diff_c, dist, src, dst, zeros)


def _final_body(ns, nv0, nv1, nv2, rs, rv0, rv1, rv2,
                os_, ov0, ov1, ov2):
    os_[...] = ns[...] + rs[0] + rs[1]
    ov0[...] = nv0[...] + rv0[0] + rv0[1]
    ov1[...] = nv1[...] + rv1[0] + rv1[1]
    ov2[...] = nv2[...] + rv2[0] + rv2[1]


def _final_add(ns, nv0, nv1, nv2, rs, rv0, rv1, rv2):
    n, h = ns.shape
    bn = 400
    grid = (n // bn,)
    row = lambda i: (i, 0)
    row3 = lambda i: (0, i, 0)
    out = jax.ShapeDtypeStruct((n, h), jnp.float32)
    return pl.pallas_call(
        _final_body,
        grid=grid,
        in_specs=[pl.BlockSpec((bn, h), row)] * 4
                 + [pl.BlockSpec((NC, bn, h), row3)] * 4,
        out_specs=[pl.BlockSpec((bn, h), row)] * 4,
        out_shape=[out] * 4,
    )(ns, nv0, nv1, nv2, rs, rv0, rv1, rv2)


def kernel(node_scalar, node_vector, edge, edge_diff, edge_dist, rbf_dist,
           W1, b1, a1, W2, b2, a2, Wf1, bf1, af1, Wf2, bf2, af2):
    n, h = node_scalar.shape
    e = edge.shape[0]
    e2 = -(-e // 5120) * 5120
    pad = e2 - e
    na = -(-(n + 1) // 128) * 128
    perm = _pair_perm(h)
    perm_w2 = np.concatenate([perm, h + np.arange(h, dtype=np.int32),
                              2 * h + perm])
    perm_wf2 = np.concatenate([perm, h + perm, 2 * h + perm])

    src = jnp.concatenate([edge[:, 0].astype(jnp.int32),
                           jnp.full((pad,), n, jnp.int32)])
    dst = jnp.concatenate([edge[:, 1].astype(jnp.int32),
                           jnp.zeros((pad,), jnp.int32)])
    dist_p = jnp.concatenate([edge_dist, jnp.ones((pad,), jnp.float32)])
    diff_p = jnp.concatenate([edge_diff, jnp.zeros((pad, 3), jnp.float32)])
    zeros_acc = jnp.zeros((na, h), jnp.float32)
    nv0 = node_vector[:, 0, :]
    nv1 = node_vector[:, 1, :]
    nv2 = node_vector[:, 2, :]
    nv0p = nv0[:, perm]
    nv1p = nv1[:, perm]
    nv2p = nv2[:, perm]
    W2p = W2[:, perm_w2]
    b2p = b2[perm_w2]
    Wf2p = Wf2[:, perm_wf2]
    bf2p = bf2[perm_wf2]
    d0 = diff_p[:, 0]
    d1 = diff_p[:, 1]
    d2 = diff_p[:, 2]
    b1r = b1.reshape(1, h)
    b2r = b2p.reshape(1, 3 * h)
    bf1r = bf1.reshape(1, h)
    bf2r = bf2p.reshape(1, 3 * h)
    a1r = a1.reshape(1, 1)
    a2r = a2.reshape(1, 1)
    af1r = af1.reshape(1, 1)
    af2r = af2.reshape(1, 1)

    tbl0, tbl1, tbl2, so2 = _node_mlp(node_scalar, nv0p, nv1p, nv2p,
                                      W1, b1r, a1r, W2p, b2r, a2r)
    fwv, fw2 = _filter_mlp(rbf_dist, Wf1, bf1r, af1r, Wf2p, bf2r, af2r, e2)
    fwv = lax.bitcast_convert_type(fwv, jnp.float32)

    rs = _sc_scalar_pass(so2, fw2, src, dst, zeros_acc).reshape(NC, na, h)
    rv0 = _sc_vec_pass(tbl0, fwv, d0, dist_p, src, dst,
                       zeros_acc).reshape(NC, na, h)
    rv1 = _sc_vec_pass(tbl1, fwv, d1, dist_p, src, dst,
                       zeros_acc).reshape(NC, na, h)
    rv2 = _sc_vec_pass(tbl2, fwv, d2, dist_p, src, dst,
                       zeros_acc).reshape(NC, na, h)

    os_, ov0, ov1, ov2 = _final_add(node_scalar, nv0, nv1, nv2,
                                    rs, rv0, rv1, rv2)
    out_vector = jnp.stack([ov0, ov1, ov2], axis=1)
    return (os_, out_vector)
